# baseline (speedup 1.0000x reference)
import functools

import jax
import jax.numpy as jnp
from jax import lax
from jax.experimental import pallas as pl
from jax.experimental.pallas import tpu as pltpu
from jax.experimental.pallas import tpu_sc as plsc

B = 4096
H = 200
HH = 100
D = 64
L = 16
ND = D // L


def kernel(indices, table):
    info = plsc.get_sparse_core_info()
    nw = info.num_cores * info.num_subcores
    bpw = B // nw
    idx2 = indices.reshape(2 * B, HH).astype(jnp.int32)

    mesh = plsc.VectorSubcoreMesh(core_axis_name="c", subcore_axis_name="s")

    @functools.partial(
        pl.kernel,
        out_type=jax.ShapeDtypeStruct((B, D), jnp.float32),
        mesh=mesh,
        compiler_params=pltpu.CompilerParams(use_tc_tiling_on_sc=False),
        scratch_types=[
            pltpu.VMEM((2 * bpw, HH), jnp.int32),
            pltpu.VMEM((H, D), jnp.float32),
            pltpu.VMEM((H, D), jnp.float32),
            pltpu.VMEM((bpw, D), jnp.float32),
            pltpu.SemaphoreType.DMA,
            pltpu.SemaphoreType.DMA,
        ],
    )
    def run(idx_hbm, tab_hbm, out_hbm, idx_v, rows0, rows1, out_v, sem0, sem1):
        wid = lax.axis_index("s") * info.num_cores + lax.axis_index("c")
        base = wid * bpw
        pltpu.sync_copy(idx_hbm.at[pl.ds(base * 2, 2 * bpw)], idx_v)

        rows = (rows0, rows1)
        sems = (sem0, sem1)

        def fire(b, k):
            pltpu.async_copy(
                tab_hbm.at[idx_v.at[2 * b]], rows[k].at[pl.ds(0, HH)], sems[k]
            )
            pltpu.async_copy(
                tab_hbm.at[idx_v.at[2 * b + 1]], rows[k].at[pl.ds(HH, HH)], sems[k]
            )

        def drain(k):
            pltpu.make_async_copy(tab_hbm.at[pl.ds(0, H)], rows[k], sems[k]).wait()

        def accum(b, rref):
            def rbody(g, acc):
                for j in range(8):
                    r = g * 8 + j
                    acc = tuple(
                        acc[d] + rref[r, pl.ds(L * d, L)] for d in range(ND)
                    )
                return acc

            acc = lax.fori_loop(
                0, H // 8, rbody,
                tuple(jnp.zeros((L,), jnp.float32) for _ in range(ND)),
            )
            for d in range(ND):
                out_v[b, pl.ds(L * d, L)] = acc[d]

        fire(0, 0)

        def body(g, carry):
            b0 = 2 * g
            fire(b0 + 1, 1)
            drain(0)
            accum(b0, rows0)

            @pl.when(b0 + 2 < bpw)
            def _():
                fire(b0 + 2, 0)

            drain(1)
            accum(b0 + 1, rows1)
            return carry

        lax.fori_loop(0, bpw // 2, body, 0)

        pltpu.sync_copy(out_v, out_hbm.at[pl.ds(base, bpw)])

    return run(idx2, table)


# speedup vs baseline: 1.0583x; 1.0583x over previous
import functools

import jax
import jax.numpy as jnp
from jax import lax
from jax.experimental import pallas as pl
from jax.experimental.pallas import tpu as pltpu
from jax.experimental.pallas import tpu_sc as plsc

B = 4096
H = 200
HH = 100
D = 64
L = 16
ND = D // L
NBUF = 4


def kernel(indices, table):
    info = plsc.get_sparse_core_info()
    nw = info.num_cores * info.num_subcores
    bpw = B // nw
    idx2 = indices.reshape(2 * B, HH).astype(jnp.int32)

    mesh = plsc.VectorSubcoreMesh(core_axis_name="c", subcore_axis_name="s")

    @functools.partial(
        pl.kernel,
        out_type=jax.ShapeDtypeStruct((B, D), jnp.float32),
        mesh=mesh,
        compiler_params=pltpu.CompilerParams(use_tc_tiling_on_sc=False),
        scratch_types=[
            pltpu.VMEM((2 * bpw, HH), jnp.int32),
            pltpu.VMEM((NBUF, H, D), jnp.float32),
            pltpu.VMEM((bpw, D), jnp.float32),
        ] + [pltpu.SemaphoreType.DMA] * NBUF,
    )
    def run(idx_hbm, tab_hbm, out_hbm, idx_v, rows_v, out_v, *sems):
        wid = lax.axis_index("s") * info.num_cores + lax.axis_index("c")
        base = wid * bpw
        pltpu.sync_copy(idx_hbm.at[pl.ds(base * 2, 2 * bpw)], idx_v)

        rows = tuple(rows_v.at[k] for k in range(NBUF))

        def fire(b, k):
            pltpu.async_copy(
                tab_hbm.at[idx_v.at[2 * b]], rows[k].at[pl.ds(0, HH)], sems[k]
            )
            pltpu.async_copy(
                tab_hbm.at[idx_v.at[2 * b + 1]], rows[k].at[pl.ds(HH, HH)], sems[k]
            )

        def drain(k):
            pltpu.make_async_copy(tab_hbm.at[pl.ds(0, H)], rows[k], sems[k]).wait()

        def accum(b, rref):
            def rbody(g, acc):
                for j in range(8):
                    r = g * 8 + j
                    acc = tuple(
                        acc[d] + rref[r, pl.ds(L * d, L)] for d in range(ND)
                    )
                return acc

            acc = lax.fori_loop(
                0, H // 8, rbody,
                tuple(jnp.zeros((L,), jnp.float32) for _ in range(ND)),
            )
            for d in range(ND):
                out_v[b, pl.ds(L * d, L)] = acc[d]

        for k in range(NBUF - 1):
            fire(k, k)

        def body(g, carry):
            b0 = NBUF * g
            for k in range(NBUF):
                b = b0 + k

                @pl.when(b + NBUF - 1 < bpw)
                def _(b=b, k=k):
                    fire(b + NBUF - 1, (k + NBUF - 1) % NBUF)

                drain(k)
                accum(b, rows[k])
            return carry

        lax.fori_loop(0, bpw // NBUF, body, 0)

        pltpu.sync_copy(out_v, out_hbm.at[pl.ds(base, bpw)])

    return run(idx2, table)
